# Initial kernel scaffold; baseline (speedup 1.0000x reference)
import jax
import jax.numpy as jnp
from jax.experimental import pallas as pl


def kernel(x, top_k_index, top_k_weights, gate_up_proj, down_proj):
    raise NotImplementedError("write your pallas kernel here")


# speedup vs baseline: 1.3551x; 1.3551x over previous
import functools

import jax
import jax.numpy as jnp
from jax import lax
from jax.experimental import pallas as pl
from jax.experimental.pallas import tpu as pltpu
from jax.experimental.pallas import tpu_sc as plsc

E = 8
D = 2048
DI = 4096
T = 4096
K = 2
S = T * K

TM = 256
NP = S + E * TM
NT = NP // TM
NB = 512
NN = DI // NB

NC, NS = 2, 16
NW = NC * NS

_SC_MESH = dict(core_axis_name="c", subcore_axis_name="s",
                num_cores=NC, num_subcores=NS)


def _routing(top_k_index, top_k_weights):
    expert = top_k_index.reshape(-1).astype(jnp.int32)
    token = jnp.arange(S, dtype=jnp.int32) // K
    order = jnp.argsort(expert, stable=True)
    sorted_expert = expert[order]
    counts = jnp.bincount(expert, length=E).astype(jnp.int32)
    group_off = jnp.concatenate(
        [jnp.zeros(1, jnp.int32), jnp.cumsum(counts)]).astype(jnp.int32)
    padded = ((counts + TM - 1) // TM) * TM
    padded_off = jnp.concatenate(
        [jnp.zeros(1, jnp.int32), jnp.cumsum(padded)]).astype(jnp.int32)
    pos = (padded_off[sorted_expert]
           + jnp.arange(S, dtype=jnp.int32) - group_off[sorted_expert])
    slot_token = jnp.zeros(NP, jnp.int32).at[pos].set(token[order])
    slot_weight = jnp.zeros(NP, jnp.float32).at[pos].set(
        top_k_weights.reshape(-1)[order])
    inv_pos = jnp.zeros(S, jnp.int32).at[order].set(pos).reshape(T, K)
    tile_expert = jnp.searchsorted(
        padded_off, jnp.arange(NT, dtype=jnp.int32) * TM,
        side="right").astype(jnp.int32) - 1
    tile_expert = jnp.clip(tile_expert, 0, E - 1)
    return slot_token, slot_weight, inv_pos, tile_expert


_G_CH = 16
_G_ROWS = NP // NW


def _gather_body(x_hbm, idx_hbm, out_hbm, idx_v, rows_v, sem):
    wid = lax.axis_index("s") * NC + lax.axis_index("c")
    base = wid * _G_ROWS

    def chunk(i, carry):
        off = base + i * _G_CH
        pltpu.sync_copy(idx_hbm.at[pl.ds(off, _G_CH)], idx_v)
        pltpu.async_copy(x_hbm.at[idx_v], rows_v, sem).wait()
        pltpu.sync_copy(rows_v, out_hbm.at[pl.ds(off, _G_CH)])
        return carry

    lax.fori_loop(0, _G_ROWS // _G_CH, chunk, 0)


def _sc_gather(x, slot_token):
    return pl.kernel(
        _gather_body,
        out_type=jax.ShapeDtypeStruct((NP, D), jnp.float32),
        mesh=plsc.VectorSubcoreMesh(**_SC_MESH),
        scratch_types=[
            pltpu.VMEM((_G_CH,), jnp.int32),
            pltpu.VMEM((_G_CH, D), jnp.float32),
            pltpu.SemaphoreType.DMA,
        ],
    )(x, slot_token)


def _ffn_body(te_ref, x_ref, g_ref, u_ref, d_ref, w_ref, out_ref):
    n = pl.program_id(1)
    xb = x_ref[...].astype(jnp.bfloat16)
    gw = g_ref[0].astype(jnp.bfloat16)
    uw = u_ref[0].astype(jnp.bfloat16)
    dw = d_ref[0].astype(jnp.bfloat16)
    g = jnp.dot(xb, gw, preferred_element_type=jnp.float32)
    u = jnp.dot(xb, uw, preferred_element_type=jnp.float32)
    h = jax.nn.gelu(g, approximate=True) * u
    p = jnp.dot(h.astype(jnp.bfloat16), dw,
                preferred_element_type=jnp.float32)
    p = p * w_ref[0, 0, :][:, None]

    @pl.when(n == 0)
    def _():
        out_ref[...] = p

    @pl.when(n != 0)
    def _():
        out_ref[...] += p


def _tc_ffn(x_g, gate_up_proj, down_proj, slot_weight, tile_expert):
    w3 = slot_weight.reshape(NT, 1, TM)
    grid_spec = pltpu.PrefetchScalarGridSpec(
        num_scalar_prefetch=1,
        grid=(NT, NN),
        in_specs=[
            pl.BlockSpec((TM, D), lambda i, n, te: (i, 0)),
            pl.BlockSpec((1, D, NB), lambda i, n, te: (te[i], 0, n)),
            pl.BlockSpec((1, D, NB), lambda i, n, te: (te[i], 0, NN + n)),
            pl.BlockSpec((1, NB, D), lambda i, n, te: (te[i], n, 0)),
            pl.BlockSpec((1, 1, TM), lambda i, n, te: (i, 0, 0)),
        ],
        out_specs=pl.BlockSpec((TM, D), lambda i, n, te: (i, 0)),
    )
    return pl.pallas_call(
        _ffn_body,
        grid_spec=grid_spec,
        out_shape=jax.ShapeDtypeStruct((NP, D), jnp.float32),
        compiler_params=pltpu.CompilerParams(
            dimension_semantics=("arbitrary", "arbitrary")),
    )(tile_expert, x_g, gate_up_proj, gate_up_proj, down_proj, w3)


_C_CH = 16
_C_TOK = T // NW
_VR = D // 16


def _combine_body(hg_hbm, p0_hbm, p1_hbm, out_hbm,
                  i0_v, i1_v, r0_v, r1_v, s0, s1):
    wid = lax.axis_index("s") * NC + lax.axis_index("c")
    base = wid * _C_TOK

    def chunk(i, carry):
        off = base + i * _C_CH
        pltpu.sync_copy(p0_hbm.at[pl.ds(off, _C_CH)], i0_v)
        pltpu.sync_copy(p1_hbm.at[pl.ds(off, _C_CH)], i1_v)
        c0 = pltpu.async_copy(hg_hbm.at[i0_v], r0_v, s0)
        c1 = pltpu.async_copy(hg_hbm.at[i1_v], r1_v, s1)
        c0.wait()
        c1.wait()

        def row(r, carry2):
            def vec(j, carry3):
                sl = pl.ds(j * 16, 16)
                r0_v[r, sl] = r0_v[r, sl] + r1_v[r, sl]
                return carry3
            return lax.fori_loop(0, _VR, vec, carry2, unroll=8)

        lax.fori_loop(0, _C_CH, row, 0)
        pltpu.sync_copy(r0_v, out_hbm.at[pl.ds(off, _C_CH)])
        return carry

    lax.fori_loop(0, _C_TOK // _C_CH, chunk, 0)


def _sc_combine(h_g, inv_pos):
    p0 = inv_pos[:, 0]
    p1 = inv_pos[:, 1]
    return pl.kernel(
        _combine_body,
        out_type=jax.ShapeDtypeStruct((T, D), jnp.float32),
        mesh=plsc.VectorSubcoreMesh(**_SC_MESH),
        scratch_types=[
            pltpu.VMEM((_C_CH,), jnp.int32),
            pltpu.VMEM((_C_CH,), jnp.int32),
            pltpu.VMEM((_C_CH, D), jnp.float32),
            pltpu.VMEM((_C_CH, D), jnp.float32),
            pltpu.SemaphoreType.DMA,
            pltpu.SemaphoreType.DMA,
        ],
    )(h_g, p0, p1)


def kernel(x, top_k_index, top_k_weights, gate_up_proj, down_proj):
    slot_token, slot_weight, inv_pos, tile_expert = _routing(
        top_k_index, top_k_weights)
    x_g = _sc_gather(x, slot_token)
    h_g = _tc_ffn(x_g, gate_up_proj, down_proj, slot_weight, tile_expert)
    return _sc_combine(h_g, inv_pos)
